# baseline (speedup 1.0000x reference)
import jax
import jax.numpy as jnp
from jax.experimental import pallas as pl
from jax.experimental.pallas import tpu as pltpu

_N_CHUNKS = 16


def _copy_body(x_ref, o_ref, sems):
    for i in range(_N_CHUNKS):
        pltpu.make_async_copy(x_ref.at[i], o_ref.at[i], sems.at[i]).start()
    for i in range(_N_CHUNKS):
        pltpu.make_async_copy(x_ref.at[i], o_ref.at[i], sems.at[i]).wait()


def kernel(inputs):
    x = inputs.reshape(_N_CHUNKS, 275, 8192)
    out = pl.pallas_call(
        _copy_body,
        out_shape=jax.ShapeDtypeStruct(x.shape, x.dtype),
        in_specs=[pl.BlockSpec(memory_space=pltpu.MemorySpace.HBM)],
        out_specs=pl.BlockSpec(memory_space=pltpu.MemorySpace.HBM),
        scratch_shapes=[pltpu.SemaphoreType.DMA((_N_CHUNKS,))],
    )(x)
    return out.reshape(inputs.shape)


# speedup vs baseline: 5.0924x; 5.0924x over previous
import jax
import jax.numpy as jnp
from jax.experimental import pallas as pl
from jax.experimental.pallas import tpu as pltpu

_ROWS = 2200
_COLS = 16384
_BLOCK_ROWS = 88


def _copy_body(x_ref, o_ref):
    o_ref[...] = x_ref[...]


def kernel(inputs):
    x = inputs.reshape(_ROWS, _COLS)
    out = pl.pallas_call(
        _copy_body,
        out_shape=jax.ShapeDtypeStruct(x.shape, x.dtype),
        grid=(_ROWS // _BLOCK_ROWS,),
        in_specs=[pl.BlockSpec((_BLOCK_ROWS, _COLS), lambda i: (i, 0))],
        out_specs=pl.BlockSpec((_BLOCK_ROWS, _COLS), lambda i: (i, 0)),
    )(x)
    return out.reshape(inputs.shape)


# speedup vs baseline: 12.9993x; 2.5527x over previous
import jax
import jax.numpy as jnp
from jax.experimental import pallas as pl
from jax.experimental.pallas import tpu as pltpu

_LEAD = 1024
_BLOCK = 32


def _copy_body(x_ref, o_ref):
    o_ref[...] = x_ref[...]


def kernel(inputs):
    x = inputs.reshape(_LEAD, 200, 176)
    out = pl.pallas_call(
        _copy_body,
        out_shape=jax.ShapeDtypeStruct(x.shape, x.dtype),
        grid=(_LEAD // _BLOCK,),
        in_specs=[pl.BlockSpec((_BLOCK, 200, 176), lambda i: (i, 0, 0))],
        out_specs=pl.BlockSpec((_BLOCK, 200, 176), lambda i: (i, 0, 0)),
    )(x)
    return out.reshape(inputs.shape)
